# Initial kernel scaffold; baseline (speedup 1.0000x reference)
import jax
import jax.numpy as jnp
from jax.experimental import pallas as pl


def kernel(input, weight):
    raise NotImplementedError("write your pallas kernel here")


# speedup vs baseline: 1.1036x; 1.1036x over previous
import functools

import jax
import jax.numpy as jnp
from jax import lax
from jax.experimental import pallas as pl
from jax.experimental.pallas import tpu as pltpu
from jax.experimental.pallas import tpu_sc as plsc

NUM_EMB = 1000000
DIM = 32
NC = 2
NS = 16
NW = NC * NS

CHUNK = 1600


def _emb_kernel(n_flat: int):
    b_per_w = n_flat // NW
    n_chunks = b_per_w // CHUNK
    mesh = plsc.VectorSubcoreMesh(
        core_axis_name="c", subcore_axis_name="s", num_cores=NC, num_subcores=NS
    )

    @functools.partial(
        pl.kernel,
        out_type=jax.ShapeDtypeStruct((n_flat, DIM), jnp.float32),
        mesh=mesh,
        scratch_types=[
            pltpu.VMEM((CHUNK,), jnp.int32),
            pltpu.VMEM((CHUNK, DIM), jnp.float32),
            pltpu.SemaphoreType.DMA,
        ],
        compiler_params=pltpu.CompilerParams(use_tc_tiling_on_sc=False),
    )
    def body(idx_hbm, table_hbm, out_hbm, idx_v, rows_v, sem):
        wid = lax.axis_index("s") * NC + lax.axis_index("c")
        base = wid * b_per_w

        def chunk(i, carry):
            off = base + i * CHUNK
            pltpu.sync_copy(idx_hbm.at[pl.ds(off, CHUNK)], idx_v)
            pltpu.async_copy(table_hbm.at[idx_v], rows_v, sem).wait()
            pltpu.sync_copy(rows_v, out_hbm.at[pl.ds(off, CHUNK)])
            return carry

        lax.fori_loop(0, n_chunks, chunk, 0)

    return body


def kernel(input, weight):
    b, h = input.shape
    flat_idx = input.reshape(b * h).astype(jnp.int32)
    out = _emb_kernel(b * h)(flat_idx, weight)
    return out.reshape(b, h, DIM)


# speedup vs baseline: 1.1134x; 1.0089x over previous
import functools

import jax
import jax.numpy as jnp
from jax import lax
from jax.experimental import pallas as pl
from jax.experimental.pallas import tpu as pltpu
from jax.experimental.pallas import tpu_sc as plsc

NUM_EMB = 1000000
DIM = 32
NC = 2
NS = 16
NW = NC * NS

CHUNK = 1280


def _emb_kernel(n_flat: int):
    b_per_w = n_flat // NW
    n_chunks = b_per_w // CHUNK
    assert n_chunks % 2 == 0
    mesh = plsc.VectorSubcoreMesh(
        core_axis_name="c", subcore_axis_name="s", num_cores=NC, num_subcores=NS
    )

    @functools.partial(
        pl.kernel,
        out_type=jax.ShapeDtypeStruct((n_flat, DIM), jnp.float32),
        mesh=mesh,
        scratch_types=[
            pltpu.VMEM((b_per_w,), jnp.int32),
            pltpu.VMEM((CHUNK, DIM), jnp.float32),
            pltpu.VMEM((CHUNK, DIM), jnp.float32),
            pltpu.SemaphoreType.DMA,
            pltpu.SemaphoreType.DMA,
        ],
        compiler_params=pltpu.CompilerParams(use_tc_tiling_on_sc=False),
    )
    def body(idx_hbm, table_hbm, out_hbm, idx_v, rows0, rows1, sem0, sem1):
        wid = lax.axis_index("s") * NC + lax.axis_index("c")
        base = wid * b_per_w

        pltpu.sync_copy(idx_hbm.at[pl.ds(base, b_per_w)], idx_v)

        def gather(g, buf, sem):
            pltpu.async_copy(
                table_hbm.at[idx_v.at[pl.ds(g * CHUNK, CHUNK)]], buf, sem
            )

        gather(0, rows0, sem0)

        bufs = (rows0, rows1)
        sems = (sem0, sem1)

        def outer(p, carry):
            for b in range(2):
                g = p * 2 + b
                cur, nxt = bufs[b], bufs[1 - b]
                csem, nsem = sems[b], sems[1 - b]

                @pl.when(g + 1 < n_chunks)
                def _():
                    gather(g + 1, nxt, nsem)

                pltpu.make_async_copy(
                    table_hbm.at[idx_v.at[pl.ds(0, CHUNK)]], cur, csem
                ).wait()
                pltpu.sync_copy(cur, out_hbm.at[pl.ds(base + g * CHUNK, CHUNK)])
            return carry

        lax.fori_loop(0, n_chunks // 2, outer, 0)

    return body


def kernel(input, weight):
    b, h = input.shape
    flat_idx = input.reshape(b * h).astype(jnp.int32)
    out = _emb_kernel(b * h)(flat_idx, weight)
    return out.reshape(b, h, DIM)


# speedup vs baseline: 1.8117x; 1.6271x over previous
import functools

import jax
import jax.numpy as jnp
from jax import lax
from jax.experimental import pallas as pl
from jax.experimental.pallas import tpu as pltpu
from jax.experimental.pallas import tpu_sc as plsc

DIM = 32
NC = 2
NS = 16
NW = NC * NS

RPC = 16


def _emb_kernel(batch: int, hist: int):
    rows_per_w = batch // NW
    n_chunks = rows_per_w // RPC
    assert n_chunks % 2 == 0
    mesh = plsc.VectorSubcoreMesh(
        core_axis_name="c", subcore_axis_name="s", num_cores=NC, num_subcores=NS
    )

    @functools.partial(
        pl.kernel,
        out_type=jax.ShapeDtypeStruct((batch, hist, DIM), jnp.float32),
        mesh=mesh,
        scratch_types=[
            pltpu.VMEM((rows_per_w, hist), jnp.int32),
            pltpu.VMEM((RPC, hist, DIM), jnp.float32),
            pltpu.VMEM((RPC, hist, DIM), jnp.float32),
            pltpu.SemaphoreType.DMA,
            pltpu.SemaphoreType.DMA,
        ],
        compiler_params=pltpu.CompilerParams(use_tc_tiling_on_sc=False),
    )
    def body(idx_hbm, table_hbm, out_hbm, idx_v, rows0, rows1, sem0, sem1):
        wid = lax.axis_index("s") * NC + lax.axis_index("c")
        base = wid * rows_per_w

        pltpu.sync_copy(idx_hbm.at[pl.ds(base, rows_per_w), :], idx_v)

        def gather(g, buf, sem):
            for r in range(RPC):
                pltpu.async_copy(
                    table_hbm.at[idx_v.at[g * RPC + r]], buf.at[r], sem
                )

        def drain(g, buf, sem):
            for r in range(RPC):
                pltpu.make_async_copy(
                    table_hbm.at[idx_v.at[g * RPC + r]], buf.at[r], sem
                ).wait()

        gather(0, rows0, sem0)

        bufs = (rows0, rows1)
        sems = (sem0, sem1)

        def outer(p, carry):
            for b in range(2):
                g = p * 2 + b
                cur, nxt = bufs[b], bufs[1 - b]
                csem, nsem = sems[b], sems[1 - b]

                @pl.when(g + 1 < n_chunks)
                def _():
                    gather(g + 1, nxt, nsem)

                drain(g, cur, csem)
                pltpu.sync_copy(
                    cur, out_hbm.at[pl.ds(base + g * RPC, RPC), :, :]
                )
            return carry

        lax.fori_loop(0, n_chunks // 2, outer, 0)

    return body


def kernel(input, weight):
    b, h = input.shape
    return _emb_kernel(b, h)(input.astype(jnp.int32), weight)
